# baseline (speedup 1.0000x reference)
import functools

import jax
import jax.numpy as jnp
from jax import lax
from jax.experimental import pallas as pl
from jax.experimental.pallas import tpu as pltpu
from jax.experimental.pallas import tpu_sc as plsc

B, S, H = 4, 2048, 2048
E, K, D = 16, 2, 64
N = B * S
TB = 2048


def _router_body(x_ref, wcat_ref, be1_ref, w2t_ref, be2_ref,
                 wc1t_ref, bc1_ref, wc2t_ref, bc2_ref, ee_ref, wb0_ref,
                 bb_ref, comb_ref):
    bf = jnp.bfloat16
    dot = lambda a, b: jnp.dot(a.astype(bf), b.astype(bf),
                               preferred_element_type=jnp.float32)
    dot_t = lambda a, b: jax.lax.dot_general(
        a.astype(bf), b.astype(bf), (((1,), (1,)), ((), ())),
        preferred_element_type=jnp.float32)

    xb = x_ref[...]
    fused = dot(xb, wcat_ref[...])
    h1 = fused[:, 0:D] + be1_ref[...]
    h1 = h1 * jax.nn.sigmoid(h1)
    emb = dot(h1, w2t_ref[...]) + be2_ref[...]

    ec = dot(ee_ref[...], wc1t_ref[...]) + bc1_ref[...]
    ec = ec * jax.nn.sigmoid(ec)
    enc = dot(ec, wc2t_ref[...]) + bc2_ref[...]

    p = dot_t(enc, wb0_ref[...])
    simT = dot_t(p, emb) + bb_ref[0, 0]

    logitsT = jnp.transpose(fused[:, D:D + E])
    comb_ref[...] = logitsT + 0.3 * simT


def _router(xf, wcat, be1, w2t, be2, wc1t, bc1, wc2t, bc2, ee, wb0, bb2,
            interpret=False):
    n = xf.shape[0]
    grid = (n // TB,)
    full = lambda shape: pl.BlockSpec(shape, lambda i: (0,) * len(shape))
    return pl.pallas_call(
        _router_body,
        grid=grid,
        in_specs=[
            pl.BlockSpec((TB, H), lambda i: (i, 0)),
            full((H, 128)), full((1, D)), full((D, D)),
            full((1, D)), full((D, D)), full((1, D)), full((D, D)),
            full((1, D)), full((E, D)), full((D, D)), full((1, 1)),
        ],
        out_specs=[pl.BlockSpec((E, TB), lambda i: (0, i))],
        out_shape=[jax.ShapeDtypeStruct((E, n), jnp.float32)],
        compiler_params=pltpu.CompilerParams(
            dimension_semantics=("parallel",)),
        interpret=interpret,
    )(xf, wcat, be1, w2t, be2, wc1t, bc1, wc2t, bc2, ee, wb0, bb2)[0]


def _tree(op, xs):
    while len(xs) > 1:
        xs = [op(xs[i], xs[i + 1]) for i in range(0, len(xs), 2)]
    return xs[0]


def _sc_route(comb, n_tok):
    info = plsc.get_sparse_core_info()
    nc, ns, nl = info.num_cores, info.num_subcores, info.num_lanes
    nw = nc * ns
    ch = n_tok // nw

    mesh = plsc.VectorSubcoreMesh(core_axis_name="c", subcore_axis_name="s")

    @functools.partial(
        pl.kernel, mesh=mesh,
        out_type=[jax.ShapeDtypeStruct((E, n_tok), jnp.float32),
                  jax.ShapeDtypeStruct((K, n_tok), jnp.float32),
                  jax.ShapeDtypeStruct((K, n_tok), jnp.int32)],
        scratch_types=[pltpu.VMEM((E, ch), jnp.float32),
                       pltpu.VMEM((E, ch), jnp.float32),
                       pltpu.VMEM((K, ch), jnp.float32),
                       pltpu.VMEM((K, ch), jnp.int32)],
    )
    def route(comb_hbm, sc_hbm, ts_hbm, ti_hbm, comb_v, sc_v, ts_v, ti_v):
        wid = lax.axis_index("s") * nc + lax.axis_index("c")
        base = wid * ch
        pltpu.sync_copy(comb_hbm.at[:, pl.ds(base, ch)], comb_v)

        def group(g, carry):
            sl = pl.ds(pl.multiple_of(g * nl, nl), nl)
            vals = [comb_v[e, sl] for e in range(E)]
            m = _tree(jnp.maximum, vals)
            exs = [jnp.exp(v - m) for v in vals]
            ssum = _tree(lambda a, b: a + b, exs)
            scs = [v / ssum for v in exs]
            for e in range(E):
                sc_v[e, sl] = scs[e]
            idxs = [jnp.full((nl,), e, jnp.int32) for e in range(E)]

            def mx(a, b):
                ge = a[0] >= b[0]
                return (jnp.where(ge, a[0], b[0]), jnp.where(ge, a[1], b[1]))

            s1, i1 = _tree(mx, list(zip(scs, idxs)))
            masked = [jnp.where(i1 == e, jnp.float32(-1.0), scs[e])
                      for e in range(E)]
            s2, i2 = _tree(mx, list(zip(masked, idxs)))
            t = jnp.exp(s2 - s1)
            p1 = 1.0 / (1.0 + t)
            ts_v[0, sl] = p1
            ts_v[1, sl] = t * p1
            ti_v[0, sl] = i1
            ti_v[1, sl] = i2
            return carry

        lax.fori_loop(0, ch // nl, group, 0)
        pltpu.sync_copy(sc_v, sc_hbm.at[:, pl.ds(base, ch)])
        pltpu.sync_copy(ts_v, ts_hbm.at[:, pl.ds(base, ch)])
        pltpu.sync_copy(ti_v, ti_hbm.at[:, pl.ds(base, ch)])

    return route(comb)


def kernel(x, gate_W, We1, be1, We2, be2, Wc1, bc1, Wc2, bc2,
           expert_embeddings, Wb, bb, interpret=False):
    xf = x.reshape(-1, H)
    wcat = jnp.concatenate(
        [We1.T, gate_W.T, jnp.zeros((H, 128 - D - E), jnp.float32)], axis=1)
    args = (wcat, be1.reshape(1, D), We2.T, be2.reshape(1, D),
            Wc1.T, bc1.reshape(1, D), Wc2.T, bc2.reshape(1, D),
            expert_embeddings, Wb[0], bb.reshape(1, 1))
    nh = N // 2
    parts = [_sc_route(_router(xf[c * nh:(c + 1) * nh], *args,
                               interpret=interpret), nh)
             for c in range(2)]
    scoresT, tsT, tiT = (jnp.concatenate(leaves, axis=1)
                         for leaves in zip(*parts))
    return (tsT.T, tiT.T, scoresT.T)


# speedup vs baseline: 3.0468x; 3.0468x over previous
import functools

import jax
import jax.numpy as jnp
from jax.experimental import pallas as pl
from jax.experimental.pallas import tpu as pltpu

B, S, H = 4, 2048, 2048
E, K, D = 16, 2, 64
N = B * S
TB = 2048


def _router_body(x_ref, wcat_ref, be1_ref, w2t_ref, be2_ref,
                 wc1t_ref, bc1_ref, wc2t_ref, bc2_ref, ee_ref, wb0_ref,
                 bb_ref, scores_ref, ts_ref, ti_ref):
    bf = jnp.bfloat16
    dot = lambda a, b: jnp.dot(a.astype(bf), b.astype(bf),
                               preferred_element_type=jnp.float32)
    dot_t = lambda a, b: jax.lax.dot_general(
        a.astype(bf), b.astype(bf), (((1,), (1,)), ((), ())),
        preferred_element_type=jnp.float32)

    xb = x_ref[...]
    fused = dot(xb, wcat_ref[...])
    h1 = fused[:, 0:D] + be1_ref[...]
    h1 = h1 * jax.nn.sigmoid(h1)
    emb = dot(h1, w2t_ref[...]) + be2_ref[...]

    ec = dot(ee_ref[...], wc1t_ref[...]) + bc1_ref[...]
    ec = ec * jax.nn.sigmoid(ec)
    enc = dot(ec, wc2t_ref[...]) + bc2_ref[...]

    p = dot_t(enc, wb0_ref[...])
    simT = dot_t(p, emb) + bb_ref[0, 0]

    logitsT = jnp.transpose(fused[:, D:D + E])
    combT = logitsT + 0.3 * simT
    m = jnp.max(combT, axis=0, keepdims=True)
    ex = jnp.exp(combT - m)
    scT = ex / jnp.sum(ex, axis=0, keepdims=True)
    scores_ref[...] = scT

    idx = jax.lax.broadcasted_iota(jnp.int32, scT.shape, 0)
    s1 = jnp.max(scT, axis=0, keepdims=True)
    i1 = jnp.min(jnp.where(scT == s1, idx, E), axis=0, keepdims=True)
    masked = jnp.where(idx == i1, -1.0, scT)
    s2 = jnp.max(masked, axis=0, keepdims=True)
    i2 = jnp.min(jnp.where(masked == s2, idx, E), axis=0, keepdims=True)

    t = jnp.exp(s2 - s1)
    p1 = 1.0 / (1.0 + t)
    io = jax.lax.broadcasted_iota(jnp.int32, (K, s1.shape[1]), 0)
    ts_ref[...] = jnp.where(io == 0, p1, t * p1)
    ti_ref[...] = jnp.where(io == 0, i1, i2)


@functools.partial(jax.jit, static_argnames=("interpret",))
def _router(xf, wcat, be1, w2t, be2, wc1t, bc1, wc2t, bc2, ee, wb0, bb2,
            interpret=False):
    grid = (N // TB,)
    full = lambda shape: pl.BlockSpec(shape, lambda i: (0,) * len(shape))
    return pl.pallas_call(
        _router_body,
        grid=grid,
        in_specs=[
            pl.BlockSpec((TB, H), lambda i: (i, 0)),
            full((H, 128)), full((1, D)), full((D, D)),
            full((1, D)), full((D, D)), full((1, D)), full((D, D)),
            full((1, D)), full((E, D)), full((D, D)), full((1, 1)),
        ],
        out_specs=[
            pl.BlockSpec((E, TB), lambda i: (0, i)),
            pl.BlockSpec((K, TB), lambda i: (0, i)),
            pl.BlockSpec((K, TB), lambda i: (0, i)),
        ],
        out_shape=[
            jax.ShapeDtypeStruct((E, N), jnp.float32),
            jax.ShapeDtypeStruct((K, N), jnp.float32),
            jax.ShapeDtypeStruct((K, N), jnp.int32),
        ],
        compiler_params=pltpu.CompilerParams(
            dimension_semantics=("parallel",)),
        interpret=interpret,
    )(xf, wcat, be1, w2t, be2, wc1t, bc1, wc2t, bc2, ee, wb0, bb2)


def kernel(x, gate_W, We1, be1, We2, be2, Wc1, bc1, Wc2, bc2,
           expert_embeddings, Wb, bb, interpret=False):
    xf = x.reshape(-1, H)
    wcat = jnp.concatenate(
        [We1.T, gate_W.T, jnp.zeros((H, 128 - D - E), jnp.float32)], axis=1)
    scoresT, tsT, tiT = _router(
        xf, wcat, be1.reshape(1, D), We2.T, be2.reshape(1, D),
        Wc1.T, bc1.reshape(1, D), Wc2.T, bc2.reshape(1, D),
        expert_embeddings, Wb[0], bb.reshape(1, 1), interpret=interpret)
    return (tsT.T, tiT.T, scoresT.T)
